# Initial kernel scaffold; baseline (speedup 1.0000x reference)
import jax
import jax.numpy as jnp
from jax.experimental import pallas as pl


def kernel(x, neighbour_indices, distancesq, W0, W1, b1):
    raise NotImplementedError("write your pallas kernel here")


# speedup vs baseline: 1.1677x; 1.1677x over previous
import functools

import jax
import jax.numpy as jnp
from jax import lax
from jax.experimental import pallas as pl
from jax.experimental.pallas import tpu as pltpu
from jax.experimental.pallas import tpu_sc as plsc

D = 128
KNN = 32
NW = 32
CH = 4
N_PAD = 10240


def _sc_accumulate(table, idx_flat, dsq_flat):
    nodes_per_w = N_PAD // NW
    steps = nodes_per_w // CH
    mesh = plsc.VectorSubcoreMesh(core_axis_name="c", subcore_axis_name="s")

    @functools.partial(
        pl.kernel,
        mesh=mesh,
        out_type=jax.ShapeDtypeStruct((N_PAD, D), jnp.float32),
        scratch_types=[
            pltpu.VMEM((CH * KNN,), jnp.int32),
            pltpu.VMEM((CH * KNN,), jnp.float32),
            pltpu.VMEM((CH * KNN, D), jnp.float32),
            pltpu.VMEM((CH, D), jnp.float32),
            pltpu.SemaphoreType.DMA,
        ],
    )
    def k(table_hbm, idx_hbm, dsq_hbm, out_hbm, idx_v, dsq_v, rows_v,
          out_v, sem):
        wid = lax.axis_index("s") * 2 + lax.axis_index("c")

        def step(s, carry):
            node_base = wid * nodes_per_w + s * CH
            flat_base = node_base * KNN
            pltpu.sync_copy(idx_hbm.at[pl.ds(flat_base, CH * KNN)], idx_v)
            pltpu.sync_copy(dsq_hbm.at[pl.ds(flat_base, CH * KNN)], dsq_v)
            pltpu.async_copy(table_hbm.at[idx_v], rows_v, sem).wait()
            for n in range(CH):
                whalves = [jnp.exp(dsq_v[pl.ds(n * KNN + h * 16, 16)] * -10.0)
                           for h in range(KNN // 16)]
                accs = [jnp.zeros((16,), jnp.float32) for _ in range(D // 16)]
                for kk in range(KNN):
                    r = n * KNN + kk
                    w = whalves[kk // 16][kk % 16]
                    for c in range(D // 16):
                        accs[c] = accs[c] + w * rows_v[r, pl.ds(c * 16, 16)]
                for c in range(D // 16):
                    out_v[n, pl.ds(c * 16, 16)] = accs[c]
            pltpu.sync_copy(out_v, out_hbm.at[pl.ds(node_base, CH)])
            return carry

        lax.fori_loop(0, steps, step, 0)

    return k(table, idx_flat, dsq_flat)


def _tc_stage1(acc, x, dsq, W0):
    blk = 256
    grid = N_PAD // blk

    def body(acc_ref, x_ref, dsq_ref, w0_ref, o_ref):
        w = jnp.exp(dsq_ref[...] * -10.0)
        wsum = jnp.sum(w, axis=1, keepdims=True)
        f = (acc_ref[...] - x_ref[...] * wsum) * (1.0 / KNN)
        f = jnp.dot(f, w0_ref[...], preferred_element_type=jnp.float32)
        o_ref[...] = jnp.where(f > 0, f, jnp.exp(f) - 1.0)

    return pl.pallas_call(
        body,
        grid=(grid,),
        in_specs=[
            pl.BlockSpec((blk, D), lambda i: (i, 0)),
            pl.BlockSpec((blk, D), lambda i: (i, 0)),
            pl.BlockSpec((blk, KNN), lambda i: (i, 0)),
            pl.BlockSpec((D, D), lambda i: (0, 0)),
        ],
        out_specs=pl.BlockSpec((blk, D), lambda i: (i, 0)),
        out_shape=jax.ShapeDtypeStruct((N_PAD, D), jnp.float32),
    )(acc, x, dsq, W0)


def _tc_stage2(acc2, f0, W1, b1):
    blk = 256
    grid = N_PAD // blk

    def body(acc_ref, f0_ref, w1_ref, b1_ref, o_ref):
        f = acc_ref[...] * (1.0 / KNN)
        f = jnp.dot(f, w1_ref[...], preferred_element_type=jnp.float32)
        f = f + b1_ref[...]
        g = jnp.where(f > 0, f, jnp.exp(f) - 1.0)
        o_ref[...] = jnp.concatenate([f0_ref[...], g], axis=1)

    return pl.pallas_call(
        body,
        grid=(grid,),
        in_specs=[
            pl.BlockSpec((blk, D), lambda i: (i, 0)),
            pl.BlockSpec((blk, D), lambda i: (i, 0)),
            pl.BlockSpec((D, D), lambda i: (0, 0)),
            pl.BlockSpec((1, D), lambda i: (0, 0)),
        ],
        out_specs=pl.BlockSpec((blk, 2 * D), lambda i: (i, 0)),
        out_shape=jax.ShapeDtypeStruct((N_PAD, 2 * D), jnp.float32),
    )(acc2, f0, W1, b1)


def kernel(x, neighbour_indices, distancesq, W0, W1, b1):
    n = x.shape[0]
    pad = N_PAD - n
    xp = jnp.pad(x, ((0, pad), (0, 0)))
    idxp = jnp.pad(neighbour_indices, ((0, pad), (0, 0)))
    dsqp = jnp.pad(distancesq, ((0, pad), (0, 0)))
    idx_flat = idxp.reshape(-1)
    dsq_flat = dsqp.reshape(-1)

    acc1 = _sc_accumulate(xp, idx_flat, dsq_flat)
    f0 = _tc_stage1(acc1, xp, dsqp, W0)
    acc2 = _sc_accumulate(f0, idx_flat, dsq_flat)
    out = _tc_stage2(acc2, f0, W1, b1.reshape(1, D))
    return out[:n]


# speedup vs baseline: 1.6387x; 1.4033x over previous
import functools

import jax
import jax.numpy as jnp
from jax import lax
from jax.experimental import pallas as pl
from jax.experimental.pallas import tpu as pltpu
from jax.experimental.pallas import tpu_sc as plsc

D = 128
KNN = 32
NW = 32
CH = 4
N_PAD = 10240


def _sc_accumulate(table, idx2d, dsq2d):
    nodes_per_w = N_PAD // NW
    steps = nodes_per_w // CH
    rps = CH * KNN
    mesh = plsc.VectorSubcoreMesh(core_axis_name="c", subcore_axis_name="s")

    @functools.partial(
        pl.kernel,
        mesh=mesh,
        out_type=jax.ShapeDtypeStruct((N_PAD, D), jnp.float32),
        scratch_types=[
            pltpu.VMEM((steps, rps), jnp.int32),
            pltpu.VMEM((steps, rps), jnp.float32),
            pltpu.VMEM((2, rps, D), jnp.float32),
            pltpu.VMEM((2, CH, D), jnp.float32),
            pltpu.SemaphoreType.DMA,
            pltpu.SemaphoreType.DMA,
            pltpu.SemaphoreType.DMA,
            pltpu.SemaphoreType.DMA,
        ],
    )
    def k(table_hbm, idx_hbm, dsq_hbm, out_hbm, idx_v, dsq_v, rows_v, out_v,
          sem_g0, sem_g1, sem_s0, sem_s1):
        wid = lax.axis_index("s") * 2 + lax.axis_index("c")
        row0 = wid * steps
        pltpu.sync_copy(idx_hbm.at[pl.ds(row0, steps)], idx_v)
        pltpu.sync_copy(dsq_hbm.at[pl.ds(row0, steps)], dsq_v)
        gsems = (sem_g0, sem_g1)
        ssems = (sem_s0, sem_s1)

        def gather_desc(s, buf):
            return pltpu.make_async_copy(table_hbm.at[idx_v.at[s]],
                                         rows_v.at[buf], gsems[buf])

        def store_desc(node_base, buf):
            return pltpu.make_async_copy(out_v.at[buf],
                                         out_hbm.at[pl.ds(node_base, CH)],
                                         ssems[buf])

        def do_half(s, buf, j):
            gather_desc(s, buf).wait()
            node_base = wid * nodes_per_w + s * CH

            @pl.when(j > 0)
            def _():
                store_desc(node_base - 2 * CH, buf).wait()

            for n in range(CH):
                whalves = [jnp.exp(dsq_v[s, pl.ds(n * KNN + h * 16, 16)]
                                   * -10.0) for h in range(KNN // 16)]
                accs = [jnp.zeros((16,), jnp.float32) for _ in range(D // 16)]
                for kk in range(KNN):
                    r = n * KNN + kk
                    w = whalves[kk // 16][kk % 16]
                    for c in range(D // 16):
                        accs[c] = accs[c] + w * rows_v[buf, r,
                                                       pl.ds(c * 16, 16)]
                for c in range(D // 16):
                    out_v[buf, n, pl.ds(c * 16, 16)] = accs[c]
            store_desc(node_base, buf).start()

        gather_desc(0, 0).start()

        def body(j, carry):
            s0 = 2 * j
            s1 = s0 + 1
            gather_desc(s1, 1).start()
            do_half(s0, 0, j)
            gather_desc(jnp.minimum(s0 + 2, steps - 1), 0).start()
            do_half(s1, 1, j)
            return carry

        lax.fori_loop(0, steps // 2, body, 0)
        last0 = wid * nodes_per_w + (steps - 2) * CH
        gather_desc(steps - 1, 0).wait()
        store_desc(last0, 0).wait()
        store_desc(last0 + CH, 1).wait()

    return k(table, idx2d, dsq2d)


def _tc_stage1(acc, x, dsq, W0):
    blk = 256
    grid = N_PAD // blk

    def body(acc_ref, x_ref, dsq_ref, w0_ref, o_ref):
        w = jnp.exp(dsq_ref[...] * -10.0)
        wsum = jnp.sum(w, axis=1, keepdims=True)
        f = (acc_ref[...] - x_ref[...] * wsum) * (1.0 / KNN)
        f = jnp.dot(f, w0_ref[...], preferred_element_type=jnp.float32)
        o_ref[...] = jnp.where(f > 0, f, jnp.exp(f) - 1.0)

    return pl.pallas_call(
        body,
        grid=(grid,),
        in_specs=[
            pl.BlockSpec((blk, D), lambda i: (i, 0)),
            pl.BlockSpec((blk, D), lambda i: (i, 0)),
            pl.BlockSpec((blk, KNN), lambda i: (i, 0)),
            pl.BlockSpec((D, D), lambda i: (0, 0)),
        ],
        out_specs=pl.BlockSpec((blk, D), lambda i: (i, 0)),
        out_shape=jax.ShapeDtypeStruct((N_PAD, D), jnp.float32),
    )(acc, x, dsq, W0)


def _tc_stage2(acc2, f0, W1, b1):
    blk = 256
    grid = N_PAD // blk

    def body(acc_ref, f0_ref, w1_ref, b1_ref, o_ref):
        f = acc_ref[...] * (1.0 / KNN)
        f = jnp.dot(f, w1_ref[...], preferred_element_type=jnp.float32)
        f = f + b1_ref[...]
        g = jnp.where(f > 0, f, jnp.exp(f) - 1.0)
        o_ref[...] = jnp.concatenate([f0_ref[...], g], axis=1)

    return pl.pallas_call(
        body,
        grid=(grid,),
        in_specs=[
            pl.BlockSpec((blk, D), lambda i: (i, 0)),
            pl.BlockSpec((blk, D), lambda i: (i, 0)),
            pl.BlockSpec((D, D), lambda i: (0, 0)),
            pl.BlockSpec((1, D), lambda i: (0, 0)),
        ],
        out_specs=pl.BlockSpec((blk, 2 * D), lambda i: (i, 0)),
        out_shape=jax.ShapeDtypeStruct((N_PAD, 2 * D), jnp.float32),
    )(acc2, f0, W1, b1)


def kernel(x, neighbour_indices, distancesq, W0, W1, b1):
    n = x.shape[0]
    pad = N_PAD - n
    xp = jnp.pad(x, ((0, pad), (0, 0)))
    idxp = jnp.pad(neighbour_indices, ((0, pad), (0, 0)))
    dsqp = jnp.pad(distancesq, ((0, pad), (0, 0)))
    idx2d = idxp.reshape(-1, CH * KNN)
    dsq2d = dsqp.reshape(-1, CH * KNN)

    acc1 = _sc_accumulate(xp, idx2d, dsq2d)
    f0 = _tc_stage1(acc1, xp, dsqp, W0)
    acc2 = _sc_accumulate(f0, idx2d, dsq2d)
    out = _tc_stage2(acc2, f0, W1, b1.reshape(1, D))
    return out[:n]


# speedup vs baseline: 6.7480x; 4.1179x over previous
import functools

import jax
import jax.numpy as jnp
from jax import lax
from jax.experimental import pallas as pl
from jax.experimental.pallas import tpu as pltpu
from jax.experimental.pallas import tpu_sc as plsc

D = 128
KNN = 32
NW = 32
CH = 4
N_PAD = 10240


def _sc_accumulate(table, idx2d, dsq2d):
    ntab = table.shape[0]
    nodes_per_w = N_PAD // NW
    steps = nodes_per_w // CH
    hr = CH * KNN // 2
    hn = CH // 2
    mesh = plsc.VectorSubcoreMesh(core_axis_name="c", subcore_axis_name="s")

    @functools.partial(
        pl.kernel,
        mesh=mesh,
        out_type=jax.ShapeDtypeStruct((N_PAD, D), jnp.float32),
        scratch_types=[
            pltpu.VMEM((steps, CH * KNN), jnp.int32),
            pltpu.VMEM((steps, CH * KNN), jnp.float32),
            pltpu.VMEM((2, hr, D), jnp.float32),
            pltpu.VMEM((2, hn, D), jnp.float32),
            pltpu.VMEM_SHARED((ntab, D), jnp.float32),
            pltpu.SemaphoreType.DMA,
            pltpu.SemaphoreType.DMA,
            pltpu.SemaphoreType.DMA,
            pltpu.SemaphoreType.DMA,
        ],
    )
    def k(table_hbm, idx_hbm, dsq_hbm, out_hbm, idx_v, dsq_v, rows_v, out_v,
          table_sp, sem_g0, sem_g1, sem_s0, sem_s1):
        wid = lax.axis_index("s") * 2 + lax.axis_index("c")
        sid = lax.axis_index("s")
        row0 = wid * steps
        chunk = -(-ntab // 16) // 8 * 8
        rem = ntab - 15 * chunk

        @pl.when(sid < 15)
        def _():
            pltpu.sync_copy(table_hbm.at[pl.ds(sid * chunk, chunk)],
                            table_sp.at[pl.ds(sid * chunk, chunk)])

        @pl.when(sid == 15)
        def _():
            pltpu.sync_copy(table_hbm.at[pl.ds(15 * chunk, rem)],
                            table_sp.at[pl.ds(15 * chunk, rem)])
        pltpu.sync_copy(idx_hbm.at[pl.ds(row0, steps)], idx_v)
        pltpu.sync_copy(dsq_hbm.at[pl.ds(row0, steps)], dsq_v)
        plsc.subcore_barrier()
        gsems = (sem_g0, sem_g1)
        ssems = (sem_s0, sem_s1)

        def gather_desc(s, h):
            return pltpu.make_async_copy(
                table_sp.at[idx_v.at[s, pl.ds(h * hr, hr)]],
                rows_v.at[h], gsems[h])

        def store_desc(node_base, h):
            return pltpu.make_async_copy(out_v.at[h],
                                         out_hbm.at[pl.ds(node_base, hn)],
                                         ssems[h])

        def do_half(s, h, j):
            gather_desc(s, h).wait()
            node_base = wid * nodes_per_w + s * CH + h * hn

            @pl.when(j > 0)
            def _():
                store_desc(node_base - CH, h).wait()

            for n in range(hn):
                base = h * hr + n * KNN
                whalves = [jnp.exp(dsq_v[s, pl.ds(base + q * 16, 16)]
                                   * -10.0) for q in range(KNN // 16)]
                accs = [jnp.zeros((16,), jnp.float32) for _ in range(D // 16)]
                for kk in range(KNN):
                    r = n * KNN + kk
                    w = whalves[kk // 16][kk % 16]
                    for c in range(D // 16):
                        accs[c] = accs[c] + w * rows_v[h, r,
                                                       pl.ds(c * 16, 16)]
                for c in range(D // 16):
                    out_v[h, n, pl.ds(c * 16, 16)] = accs[c]
            store_desc(node_base, h).start()

        gather_desc(0, 0).start()

        def body(j, carry):
            gather_desc(j, 1).start()
            do_half(j, 0, j)
            gather_desc(jnp.minimum(j + 1, steps - 1), 0).start()
            do_half(j, 1, j)
            return carry

        lax.fori_loop(0, steps, body, 0)
        lastn = wid * nodes_per_w + (steps - 1) * CH
        gather_desc(steps - 1, 0).wait()
        store_desc(lastn, 0).wait()
        store_desc(lastn + hn, 1).wait()

    return k(table, idx2d, dsq2d)


def _tc_stage1(acc, x, dsq, W0):
    blk = 256
    grid = N_PAD // blk

    def body(acc_ref, x_ref, dsq_ref, w0_ref, o_ref):
        w = jnp.exp(dsq_ref[...] * -10.0)
        wsum = jnp.sum(w, axis=1, keepdims=True)
        f = (acc_ref[...] - x_ref[...] * wsum) * (1.0 / KNN)
        f = jnp.dot(f, w0_ref[...], preferred_element_type=jnp.float32)
        o_ref[...] = jnp.where(f > 0, f, jnp.exp(f) - 1.0)

    return pl.pallas_call(
        body,
        grid=(grid,),
        in_specs=[
            pl.BlockSpec((blk, D), lambda i: (i, 0)),
            pl.BlockSpec((blk, D), lambda i: (i, 0)),
            pl.BlockSpec((blk, KNN), lambda i: (i, 0)),
            pl.BlockSpec((D, D), lambda i: (0, 0)),
        ],
        out_specs=pl.BlockSpec((blk, D), lambda i: (i, 0)),
        out_shape=jax.ShapeDtypeStruct((N_PAD, D), jnp.float32),
    )(acc, x, dsq, W0)


def _tc_stage2(acc2, f0, W1, b1, n):
    blk = 256
    grid = N_PAD // blk

    def body(acc_ref, f0_ref, w1_ref, b1_ref, o_ref):
        f = acc_ref[...] * (1.0 / KNN)
        f = jnp.dot(f, w1_ref[...], preferred_element_type=jnp.float32)
        f = f + b1_ref[...]
        g = jnp.where(f > 0, f, jnp.exp(f) - 1.0)
        o_ref[...] = jnp.concatenate([f0_ref[...], g], axis=1)

    return pl.pallas_call(
        body,
        grid=(grid,),
        in_specs=[
            pl.BlockSpec((blk, D), lambda i: (i, 0)),
            pl.BlockSpec((blk, D), lambda i: (i, 0)),
            pl.BlockSpec((D, D), lambda i: (0, 0)),
            pl.BlockSpec((1, D), lambda i: (0, 0)),
        ],
        out_specs=pl.BlockSpec((blk, 2 * D), lambda i: (i, 0)),
        out_shape=jax.ShapeDtypeStruct((n, 2 * D), jnp.float32),
    )(acc2, f0, W1, b1)


def kernel(x, neighbour_indices, distancesq, W0, W1, b1):
    n = x.shape[0]
    pad = N_PAD - n
    idxp = jnp.pad(neighbour_indices, ((0, pad), (0, 0)))
    dsqp = jnp.pad(distancesq, ((0, pad), (0, 0)))
    idx2d = idxp.reshape(-1, CH * KNN)
    dsq2d = dsqp.reshape(-1, CH * KNN)

    acc1 = _sc_accumulate(x, idx2d, dsq2d)
    f0 = _tc_stage1(acc1, x, distancesq, W0)
    acc2 = _sc_accumulate(f0, idx2d, dsq2d)
    return _tc_stage2(acc2, f0, W1, b1.reshape(1, D), n)
